# baseline (speedup 1.0000x reference)
import functools

import jax
import jax.numpy as jnp
import numpy as np
from jax import lax
from jax.experimental import pallas as pl

D = 768
H = 12
DH = D // H
CHUNK = 128
DEPTH = 2
MEL = 80
PRE_H = 256
FF = 3072
S = 2048
SK = 512
NC = S // CHUNK

_SCALE = 1.0 / float(np.sqrt(DH))


def _posenc(s, d):
    pos = np.arange(s)[:, None].astype(np.float32)
    i = np.arange(d // 2)[None, :].astype(np.float32)
    ang = pos / np.power(10000.0, 2.0 * i / d)
    pe = np.zeros((s, d), dtype=np.float32)
    pe[:, 0::2] = np.sin(ang)
    pe[:, 1::2] = np.cos(ang)
    return jnp.asarray(pe)


_PE = _posenc(S, D)


def _ln_f(x):
    m = jnp.mean(x, axis=-1, keepdims=True)
    c = x - m
    v = jnp.mean(c * c, axis=-1, keepdims=True)
    return c * lax.rsqrt(v + 1e-5)


def _dot(a, b):
    return jnp.dot(a, b, preferred_element_type=jnp.float32)


_TP = 256


def _prenet_k(x_ref, w1_ref, b1_ref, w2_ref, b2_ref, pe_ref, o_ref):
    h = jnp.maximum(_dot(x_ref[...], w1_ref[...]) + b1_ref[...], 0.0)
    h = jnp.maximum(_dot(h, w2_ref[...]) + b2_ref[...], 0.0)
    o_ref[...] = h + pe_ref[...]


def _prenet(x, w1, b1, w2, b2, pe):
    return pl.pallas_call(
        _prenet_k,
        grid=(S // _TP,),
        in_specs=[
            pl.BlockSpec((_TP, MEL), lambda i: (i, 0)),
            pl.BlockSpec((MEL, PRE_H), lambda i: (0, 0)),
            pl.BlockSpec((1, PRE_H), lambda i: (0, 0)),
            pl.BlockSpec((PRE_H, D), lambda i: (0, 0)),
            pl.BlockSpec((1, D), lambda i: (0, 0)),
            pl.BlockSpec((_TP, D), lambda i: (i, 0)),
        ],
        out_specs=pl.BlockSpec((_TP, D), lambda i: (i, 0)),
        out_shape=jax.ShapeDtypeStruct((S, D), jnp.float32),
    )(x, w1, b1, w2, b2, pe)


_TL = 256


def _lnproj3_k(x_ref, w_ref, q_ref, k_ref, v_ref):
    y = _ln_f(x_ref[...])
    q_ref[...] = _dot(y, w_ref[0])
    k_ref[...] = _dot(y, w_ref[1])
    v_ref[...] = _dot(y, w_ref[2])


def _lnproj3(x, w3):
    return pl.pallas_call(
        _lnproj3_k,
        grid=(S // _TL,),
        in_specs=[
            pl.BlockSpec((_TL, D), lambda i: (i, 0)),
            pl.BlockSpec((3, D, D), lambda i: (0, 0, 0)),
        ],
        out_specs=[
            pl.BlockSpec((_TL, D), lambda i: (i, 0)),
            pl.BlockSpec((_TL, D), lambda i: (i, 0)),
            pl.BlockSpec((_TL, D), lambda i: (i, 0)),
        ],
        out_shape=[
            jax.ShapeDtypeStruct((S, D), jnp.float32),
            jax.ShapeDtypeStruct((S, D), jnp.float32),
            jax.ShapeDtypeStruct((S, D), jnp.float32),
        ],
    )(x, w3)


def _lnproj1_k(x_ref, w_ref, q_ref):
    q_ref[...] = _dot(_ln_f(x_ref[...]), w_ref[...])


def _lnproj1(x, w):
    return pl.pallas_call(
        _lnproj1_k,
        grid=(S // _TL,),
        in_specs=[
            pl.BlockSpec((_TL, D), lambda i: (i, 0)),
            pl.BlockSpec((D, D), lambda i: (0, 0)),
        ],
        out_specs=pl.BlockSpec((_TL, D), lambda i: (i, 0)),
        out_shape=jax.ShapeDtypeStruct((S, D), jnp.float32),
    )(x, w)


def _kvproj_k(x_ref, w_ref, o_ref):
    o_ref[0] = _dot(x_ref[...], w_ref[0])


def _kvproj(keys, w2):
    return pl.pallas_call(
        _kvproj_k,
        grid=(2,),
        in_specs=[
            pl.BlockSpec((SK, D), lambda j: (0, 0)),
            pl.BlockSpec((1, D, D), lambda j: (j, 0, 0)),
        ],
        out_specs=pl.BlockSpec((1, SK, D), lambda j: (j, 0, 0)),
        out_shape=jax.ShapeDtypeStruct((2, SK, D), jnp.float32),
    )(keys, w2)


def _self_attn_k(x_ref, q_ref, kp_ref, kc_ref, vp_ref, vc_ref, wo_ref, o_ref):
    i = pl.program_id(0)
    h = pl.program_id(1)

    @pl.when(h == 0)
    def _():
        o_ref[...] = x_ref[...]

    q = q_ref[0]
    k = jnp.concatenate([kp_ref[0], kc_ref[0]], axis=0)
    v = jnp.concatenate([vp_ref[0], vc_ref[0]], axis=0)
    logits = _dot(q, k.T) * _SCALE
    r = lax.broadcasted_iota(jnp.int32, (CHUNK, 2 * CHUNK), 0)
    c = lax.broadcasted_iota(jnp.int32, (CHUNK, 2 * CHUNK), 1)
    prev_ok = jnp.logical_and(c < CHUNK, i > 0)
    cur_ok = jnp.logical_and(c >= CHUNK, r >= (c - CHUNK))
    mask = jnp.logical_or(prev_ok, cur_ok)
    logits = jnp.where(mask, logits, -1e9)
    p = jax.nn.softmax(logits, axis=-1)
    a = _dot(p, v)
    o_ref[...] += _dot(a, wo_ref[...])


def _self_attn(x, q, k, v, wo):
    prev = lambda i, h: (h, jnp.maximum(i - 1, 0), 0)
    cur = lambda i, h: (h, i, 0)
    return pl.pallas_call(
        _self_attn_k,
        grid=(NC, H),
        in_specs=[
            pl.BlockSpec((CHUNK, D), lambda i, h: (i, 0)),
            pl.BlockSpec((1, CHUNK, DH), cur),
            pl.BlockSpec((1, CHUNK, DH), prev),
            pl.BlockSpec((1, CHUNK, DH), cur),
            pl.BlockSpec((1, CHUNK, DH), prev),
            pl.BlockSpec((1, CHUNK, DH), cur),
            pl.BlockSpec((DH, D), lambda i, h: (h, 0)),
        ],
        out_specs=pl.BlockSpec((CHUNK, D), lambda i, h: (i, 0)),
        out_shape=jax.ShapeDtypeStruct((S, D), jnp.float32),
    )(x, q, k, k, v, v, wo)


_TC = 512


def _cross_attn_k(x_ref, q_ref, k_ref, v_ref, wo_ref, o_ref):
    h = pl.program_id(1)

    @pl.when(h == 0)
    def _():
        o_ref[...] = x_ref[...]

    logits = _dot(q_ref[0], k_ref[0, 0].T) * _SCALE
    p = jax.nn.softmax(logits, axis=-1)
    a = _dot(p, v_ref[0, 0])
    o_ref[...] += _dot(a, wo_ref[...])


def _cross_attn(x, q, kv, wo):
    return pl.pallas_call(
        _cross_attn_k,
        grid=(S // _TC, H),
        in_specs=[
            pl.BlockSpec((_TC, D), lambda i, h: (i, 0)),
            pl.BlockSpec((1, _TC, DH), lambda i, h: (h, i, 0)),
            pl.BlockSpec((1, 1, SK, DH), lambda i, h: (0, h, 0, 0)),
            pl.BlockSpec((1, 1, SK, DH), lambda i, h: (1, h, 0, 0)),
            pl.BlockSpec((DH, D), lambda i, h: (h, 0)),
        ],
        out_specs=pl.BlockSpec((_TC, D), lambda i, h: (i, 0)),
        out_shape=jax.ShapeDtypeStruct((S, D), jnp.float32),
    )(x, q, kv, kv, wo)


_TF = 256


def _ffn_k(x_ref, w1_ref, w2_ref, o_ref):
    x = x_ref[...]
    y = _ln_f(x)
    hmid = jnp.maximum(_dot(y, w1_ref[...]), 0.0)
    o_ref[...] = x + _dot(hmid, w2_ref[...])


def _ffn(x, w1, w2):
    return pl.pallas_call(
        _ffn_k,
        grid=(S // _TF,),
        in_specs=[
            pl.BlockSpec((_TF, D), lambda i: (i, 0)),
            pl.BlockSpec((D, FF), lambda i: (0, 0)),
            pl.BlockSpec((FF, D), lambda i: (0, 0)),
        ],
        out_specs=pl.BlockSpec((_TF, D), lambda i: (i, 0)),
        out_shape=jax.ShapeDtypeStruct((S, D), jnp.float32),
    )(x, w1, w2)


_TH = 256


def _heads_k(x_ref, mw_ref, mb_ref, sw_ref, sb_ref, mel_ref, stop_ref):
    y = _ln_f(x_ref[...])
    mel_ref[...] = _dot(y, mw_ref[...]) + mb_ref[...]
    sl = _dot(y, sw_ref[...]) + sb_ref[...]
    stop_ref[...] = jax.nn.softmax(sl, axis=-1)


def _heads(x, mel_w, mel_b, stop_w, stop_b):
    return pl.pallas_call(
        _heads_k,
        grid=(S // _TH,),
        in_specs=[
            pl.BlockSpec((_TH, D), lambda i: (i, 0)),
            pl.BlockSpec((D, MEL), lambda i: (0, 0)),
            pl.BlockSpec((1, MEL), lambda i: (0, 0)),
            pl.BlockSpec((D, 2), lambda i: (0, 0)),
            pl.BlockSpec((1, 2), lambda i: (0, 0)),
        ],
        out_specs=[
            pl.BlockSpec((_TH, MEL), lambda i: (i, 0)),
            pl.BlockSpec((_TH, 2), lambda i: (i, 0)),
        ],
        out_shape=[
            jax.ShapeDtypeStruct((S, MEL), jnp.float32),
            jax.ShapeDtypeStruct((S, 2), jnp.float32),
        ],
    )(x, mel_w, mel_b, stop_w, stop_b)


_TCV = 512


def _conv_k(xm_ref, xc_ref, xp_ref, w_ref, b_ref, o_ref, *, act, cin, cout):
    i = pl.program_id(0)
    n = pl.num_programs(0)
    top = jnp.where(i > 0, xm_ref[pl.ds(_TCV - 2, 2), :], 0.0)
    bot = jnp.where(i < n - 1, xp_ref[pl.ds(0, 2), :], 0.0)
    win = jnp.concatenate([top, xc_ref[...], bot], axis=0)
    acc = jnp.broadcast_to(b_ref[...], (_TCV, cout)).astype(jnp.float32)
    for d in range(5):
        acc = acc + _dot(win[d:d + _TCV], w_ref[d])
    o_ref[...] = jnp.tanh(acc) if act else acc


def _conv(x, w, b, act):
    cin = x.shape[1]
    cout = w.shape[2]
    kfn = functools.partial(_conv_k, act=act, cin=cin, cout=cout)
    prev = lambda i: (jnp.maximum(i - 1, 0), 0)
    nxt = lambda i: (jnp.minimum(i + 1, S // _TCV - 1), 0)
    return pl.pallas_call(
        kfn,
        grid=(S // _TCV,),
        in_specs=[
            pl.BlockSpec((_TCV, cin), prev),
            pl.BlockSpec((_TCV, cin), lambda i: (i, 0)),
            pl.BlockSpec((_TCV, cin), nxt),
            pl.BlockSpec((5, cin, cout), lambda i: (0, 0, 0)),
            pl.BlockSpec((1, cout), lambda i: (0, 0)),
        ],
        out_specs=pl.BlockSpec((_TCV, cout), lambda i: (i, 0)),
        out_shape=jax.ShapeDtypeStruct((S, cout), jnp.float32),
    )(x, x, x, w, b)


def _conv_final_k(xm_ref, xc_ref, xp_ref, w_ref, b_ref, mel_ref, o_ref):
    i = pl.program_id(0)
    n = pl.num_programs(0)
    top = jnp.where(i > 0, xm_ref[pl.ds(_TCV - 2, 2), :], 0.0)
    bot = jnp.where(i < n - 1, xp_ref[pl.ds(0, 2), :], 0.0)
    win = jnp.concatenate([top, xc_ref[...], bot], axis=0)
    acc = mel_ref[...] + b_ref[...]
    for d in range(5):
        acc = acc + _dot(win[d:d + _TCV], w_ref[d])
    o_ref[...] = acc


def _conv_final(x, w, b, mel):
    prev = lambda i: (jnp.maximum(i - 1, 0), 0)
    nxt = lambda i: (jnp.minimum(i + 1, S // _TCV - 1), 0)
    return pl.pallas_call(
        _conv_final_k,
        grid=(S // _TCV,),
        in_specs=[
            pl.BlockSpec((_TCV, D), prev),
            pl.BlockSpec((_TCV, D), lambda i: (i, 0)),
            pl.BlockSpec((_TCV, D), nxt),
            pl.BlockSpec((5, D, MEL), lambda i: (0, 0, 0)),
            pl.BlockSpec((1, MEL), lambda i: (0, 0)),
            pl.BlockSpec((_TCV, MEL), lambda i: (i, 0)),
        ],
        out_specs=pl.BlockSpec((_TCV, MEL), lambda i: (i, 0)),
        out_shape=jax.ShapeDtypeStruct((S, MEL), jnp.float32),
    )(x, x, x, w, b, mel)


def kernel(input_, keys, pre_w1, pre_b1, pre_w2, pre_b2, alpha, self_qkvo,
           cross_qkvo, ff_w1, ff_w2, mel_w, mel_b, stop_w, stop_b,
           pc_w_in, pc_b_in, pc_w_mid, pc_b_mid, pc_w_out, pc_b_out):
    x_in = input_[0]
    keys0 = keys[0]
    pe = alpha * _PE

    x = _prenet(x_in, pre_w1, pre_b1[None], pre_w2, pre_b2[None], pe)

    heads_of = lambda a: a.reshape(-1, H, DH).transpose(1, 0, 2)
    for l in range(DEPTH):
        q, k, v = _lnproj3(x, self_qkvo[l, 0:3])
        x = _self_attn(x, heads_of(q), heads_of(k), heads_of(v),
                       self_qkvo[l, 3])
        qc = _lnproj1(x, cross_qkvo[l, 0])
        kv = _kvproj(keys0, cross_qkvo[l, 1:3])
        kvh = kv.reshape(2, SK, H, DH).transpose(0, 2, 1, 3)
        x = _cross_attn(x, heads_of(qc), kvh, cross_qkvo[l, 3])
        x = _ffn(x, ff_w1[l], ff_w2[l])

    mel_lin, stop = _heads(x, mel_w, mel_b[None], stop_w, stop_b[None])

    t = _conv(mel_lin, pc_w_in, pc_b_in[None], act=True)
    for j in range(3):
        t = _conv(t, pc_w_mid[j], pc_b_mid[j][None], act=True)
    mel = _conv_final(t, pc_w_out, pc_b_out[None], mel_lin)

    return (mel[None], stop[None])
